# baseline (speedup 1.0000x reference)
import functools

import jax
import jax.numpy as jnp
from jax import lax
from jax.experimental import pallas as pl
from jax.experimental.pallas import tpu as pltpu
from jax.experimental.pallas import tpu_sc as plsc

_N = 10000
_E = 160000
_D = 256
_DK = 64

_NC = 2
_NS = 16
_L = 16

_CH = 64
_G = _E // _CH
_CPS = -(-_G // _NS)
_PAIRS = -(-_CPS // 2)
_RB = 16
_NB = _N // _RB
_BPS = -(-_NB // _NS)


def _proj_body(x_ref, w_ref, b_ref, o_ref):
    o_ref[...] = (
        jnp.dot(x_ref[...], w_ref[...], preferred_element_type=jnp.float32)
        + b_ref[...]
    )


def _project(x, wcat, bcat):
    blk = 1000
    return pl.pallas_call(
        _proj_body,
        grid=(_N // blk,),
        in_specs=[
            pl.BlockSpec((blk, _D), lambda i: (i, 0)),
            pl.BlockSpec((_D, 3 * _D), lambda i: (0, 0)),
            pl.BlockSpec((1, 3 * _D), lambda i: (0, 0)),
        ],
        out_specs=pl.BlockSpec((blk, 3 * _D), lambda i: (i, 0)),
        out_shape=jax.ShapeDtypeStruct((_N, 3 * _D), jnp.float32),
    )(x, wcat, bcat)


def _sc_attention(epk, q0, kv0, q1, kv1):
    mesh = plsc.VectorSubcoreMesh(
        core_axis_name="c", subcore_axis_name="s",
        num_cores=_NC, num_subcores=_NS,
    )

    buf_set = [
        pltpu.VMEM((2, _CH), jnp.int32),
        pltpu.VMEM((_CH, 2 * _DK), jnp.float32),
        pltpu.VMEM((_CH, 4 * _DK), jnp.float32),
        pltpu.SemaphoreType.DMA,
        pltpu.SemaphoreType.DMA,
        pltpu.SemaphoreType.DMA,
    ]

    @functools.partial(
        pl.kernel,
        out_type=[
            jax.ShapeDtypeStruct((_N, 2 * _DK), jnp.float32),
            jax.ShapeDtypeStruct((_N, 2 * _DK), jnp.float32),
        ],
        mesh=mesh,
        scratch_types=buf_set + buf_set + [
            pltpu.VMEM_SHARED((_N, 2 * _DK), jnp.float32),
            pltpu.VMEM_SHARED((_N,), jnp.float32),
            pltpu.VMEM_SHARED((_N,), jnp.float32),
        ],
        compiler_params=pltpu.CompilerParams(needs_layout_passes=False),
    )
    def attn(epk_h, q0_h, kv0_h, q1_h, kv1_h, out0_h, out1_h,
             ia0, qra0, kva0, gsa0, ssa0, isa0,
             ia1, qra1, kva1, gsa1, ssa1, isa1,
             accum, dacc0, dacc1):
        cid = lax.axis_index("c")
        sid = lax.axis_index("s")
        lane = lax.iota(jnp.int32, _L)
        zeros = jnp.zeros((_L,), jnp.float32)
        sets = (
            (ia0, qra0, kva0, gsa0, ssa0, isa0),
            (ia1, qra1, kva1, gsa1, ssa1, isa1),
        )

        for r in range(_RB):
            for j in range(8):
                qra0[r, j * _L:(j + 1) * _L] = zeros

        def _zero_blk(t, _):
            b = sid + _NS * t

            @pl.when(b < _NB)
            def _():
                pltpu.sync_copy(qra0.at[pl.ds(0, _RB)], accum.at[pl.ds(b * _RB, _RB)])
                pltpu.sync_copy(qra0.at[0, pl.ds(0, _RB)], dacc0.at[pl.ds(b * _RB, _RB)])
                pltpu.sync_copy(qra0.at[0, pl.ds(0, _RB)], dacc1.at[pl.ds(b * _RB, _RB)])

            return _

        lax.fori_loop(0, _BPS, _zero_blk, None)
        plsc.subcore_barrier()

        def _run(qt, kvt):
            def fire_idx(S, i):
                idxb, isem = S[0], S[5]
                g = sid + _NS * i
                pltpu.async_copy(epk_h.at[g], idxb, isem)

            def wait_idx(S):
                idxb, isem = S[0], S[5]
                pltpu.make_async_copy(epk_h.at[0], idxb, isem).wait()

            def fire_gather(S):
                idxb, qr, kvr, gsem = S[:4]
                pltpu.async_copy(qt.at[idxb.at[0]], qr, gsem)
                pltpu.async_copy(kvt.at[idxb.at[1]], kvr, gsem)

            def drain_gather(S):
                idxb, qr, kvr, gsem = S[:4]
                pltpu.make_async_copy(qt.at[idxb.at[0]], qr, gsem).wait()
                pltpu.make_async_copy(kvt.at[idxb.at[1]], kvr, gsem).wait()

            def fire_scatter(S):
                idxb, qr, kvr = S[:3]
                ssem = S[4]
                pltpu.async_copy(qr, accum.at[idxb.at[0]], ssem, add=True)
                pltpu.async_copy(
                    kvr.at[0, pl.ds(0, _CH)], dacc0.at[idxb.at[0]],
                    ssem, add=True)
                pltpu.async_copy(
                    kvr.at[0, pl.ds(_CH, _CH)], dacc1.at[idxb.at[0]],
                    ssem, add=True)

            def drain_scatter(S):
                idxb, qr, kvr = S[:3]
                ssem = S[4]
                pltpu.make_async_copy(qr, accum.at[idxb.at[0]], ssem).wait()
                pltpu.make_async_copy(
                    kvr.at[0, pl.ds(0, _CH)], dacc0.at[idxb.at[0]],
                    ssem).wait()
                pltpu.make_async_copy(
                    kvr.at[0, pl.ds(_CH, _CH)], dacc1.at[idxb.at[0]],
                    ssem).wait()

            def compute(S):
                _, qr, kvr = S[:3]

                def _edge(e, _):
                    acc0 = qr[e, 0:_L] * kvr[e, 0:_L]
                    acc1 = qr[e, 4 * _L:5 * _L] * kvr[e, 4 * _L:5 * _L]
                    for j in range(1, 4):
                        acc0 = acc0 + qr[e, j * _L:(j + 1) * _L] * kvr[e, j * _L:(j + 1) * _L]
                        jj = j + 4
                        acc1 = acc1 + qr[e, jj * _L:(jj + 1) * _L] * kvr[e, jj * _L:(jj + 1) * _L]
                    w0 = jnp.exp(jnp.full((_L,), jnp.sum(acc0), jnp.float32))
                    w1 = jnp.exp(jnp.full((_L,), jnp.sum(acc1), jnp.float32))
                    for j in range(4):
                        qr[e, j * _L:(j + 1) * _L] = w0 * kvr[e, (8 + j) * _L:(9 + j) * _L]
                    for j in range(4, 8):
                        qr[e, j * _L:(j + 1) * _L] = w1 * kvr[e, (8 + j) * _L:(9 + j) * _L]
                    plsc.store_compressed(kvr.at[0, pl.ds(e, _L)], w0, mask=lane == 0)
                    plsc.store_compressed(kvr.at[0, pl.ds(_CH + e, _L)], w1, mask=lane == 0)
                    return _

                lax.fori_loop(0, _CH, _edge, None, unroll=4)

            fire_idx(sets[0], 0)
            wait_idx(sets[0])
            fire_gather(sets[0])

            def _pair(t, _):
                for b in (0, 1):
                    S = sets[b]
                    T = sets[1 - b]
                    i = 2 * t + b
                    g = sid + _NS * i

                    @pl.when(g < _G)
                    def _():
                        drain_gather(S)

                    @pl.when(sid + _NS * (i + 1) < _G)
                    def _():
                        @pl.when(i >= 1)
                        def _():
                            drain_scatter(T)

                        fire_idx(T, i + 1)
                        wait_idx(T)
                        fire_gather(T)

                    @pl.when(g < _G)
                    def _():
                        compute(S)
                        fire_scatter(S)

                return _

            lax.fori_loop(0, _PAIRS, _pair, None)
            drain_scatter(sets[0])
            drain_scatter(sets[1])

        @pl.when(cid == 0)
        def _():
            _run(q0_h, kv0_h)

        @pl.when(cid == 1)
        def _():
            _run(q1_h, kv1_h)

        plsc.subcore_barrier()

        def _fin(t, _):
            b = sid + _NS * t

            @pl.when(b < _NB)
            def _():
                pltpu.sync_copy(accum.at[pl.ds(b * _RB, _RB)], qra0.at[pl.ds(0, _RB)])
                pltpu.sync_copy(dacc0.at[pl.ds(b * _RB, _RB)], qra0.at[16, pl.ds(0, _RB)])
                pltpu.sync_copy(dacc1.at[pl.ds(b * _RB, _RB)], qra0.at[16, pl.ds(_L, _RB)])
                dv0 = 1.0 / (qra0[16, 0:_L] + 1e-16)
                dv1 = 1.0 / (qra0[16, _L:2 * _L] + 1e-16)
                for r in range(_RB):
                    i0 = jnp.full((_L,), dv0[r], jnp.float32)
                    i1 = jnp.full((_L,), dv1[r], jnp.float32)
                    for j in range(4):
                        qra0[24 + r, j * _L:(j + 1) * _L] = qra0[r, j * _L:(j + 1) * _L] * i0
                    for j in range(4, 8):
                        qra0[24 + r, j * _L:(j + 1) * _L] = qra0[r, j * _L:(j + 1) * _L] * i1

                @pl.when(cid == 0)
                def _():
                    pltpu.sync_copy(qra0.at[pl.ds(24, _RB)],
                                    out0_h.at[pl.ds(b * _RB, _RB)])

                @pl.when(cid == 1)
                def _():
                    pltpu.sync_copy(qra0.at[pl.ds(24, _RB)],
                                    out1_h.at[pl.ds(b * _RB, _RB)])

            return _

        lax.fori_loop(0, _BPS, _fin, None)

    return attn(epk, q0, kv0, q1, kv1)


def kernel(x, edge, Qw, Qb, Kw, Kb, Vw, Vb):
    scale = 1.0 / (_DK ** 0.5)
    wcat = jnp.concatenate([Qw, Kw * scale, Vw], axis=1)
    bcat = jnp.concatenate([Qb, Kb * scale, Vb]).reshape(1, 3 * _D)
    qkv = _project(x, wcat, bcat)
    q0 = qkv[:, 0:128]
    q1 = qkv[:, 128:256]
    kv0 = jnp.concatenate([qkv[:, 256:384], qkv[:, 512:640]], axis=1)
    kv1 = jnp.concatenate([qkv[:, 384:512], qkv[:, 640:768]], axis=1)
    epk = edge.reshape(2, _G, _CH).transpose(1, 0, 2)
    o0, o1 = _sc_attention(epk, q0, kv0, q1, kv1)
    return jnp.concatenate([o0, o1], axis=1)


# speedup vs baseline: 1.5229x; 1.5229x over previous
import functools

import jax
import jax.numpy as jnp
from jax import lax
from jax.experimental import pallas as pl
from jax.experimental.pallas import tpu as pltpu
from jax.experimental.pallas import tpu_sc as plsc

_N = 10000
_E = 160000
_D = 256
_DK = 64

_NC = 2
_NS = 16
_L = 16

_CH = 64
_G = _E // _CH
_CPS = -(-_G // _NS)
_PAIRS = -(-_CPS // 2)
_RB = 16
_NB = _N // _RB
_BPS = -(-_NB // _NS)


def _proj_body(x_ref, w_ref, b_ref, o_ref):
    o_ref[...] = (
        jnp.dot(x_ref[...], w_ref[...], preferred_element_type=jnp.float32)
        + b_ref[...]
    )


def _project(x, wcat, bcat):
    blk = 1000
    return pl.pallas_call(
        _proj_body,
        grid=(_N // blk,),
        in_specs=[
            pl.BlockSpec((blk, _D), lambda i: (i, 0)),
            pl.BlockSpec((_D, 3 * _D), lambda i: (0, 0)),
            pl.BlockSpec((1, 3 * _D), lambda i: (0, 0)),
        ],
        out_specs=pl.BlockSpec((blk, 3 * _D), lambda i: (i, 0)),
        out_shape=jax.ShapeDtypeStruct((_N, 3 * _D), jnp.float32),
    )(x, wcat, bcat)


def _sc_attention(edge, q0, k0, v0, q1, k1, v1):
    mesh = plsc.VectorSubcoreMesh(
        core_axis_name="c", subcore_axis_name="s",
        num_cores=_NC, num_subcores=_NS,
    )

    buf_set = [
        pltpu.VMEM((_CH,), jnp.int32),
        pltpu.VMEM((_CH,), jnp.int32),
        pltpu.VMEM((_CH, 2 * _DK), jnp.float32),
        pltpu.VMEM((_CH, 2 * _DK), jnp.float32),
        pltpu.VMEM((_CH, 2 * _DK), jnp.float32),
        pltpu.SemaphoreType.DMA,
        pltpu.SemaphoreType.DMA,
        pltpu.SemaphoreType.DMA,
    ]

    @functools.partial(
        pl.kernel,
        out_type=[
            jax.ShapeDtypeStruct((_N, 2 * _DK), jnp.float32),
            jax.ShapeDtypeStruct((_N, 2 * _DK), jnp.float32),
        ],
        mesh=mesh,
        scratch_types=buf_set + buf_set + [
            pltpu.VMEM_SHARED((_N, 2 * _DK), jnp.float32),
            pltpu.VMEM_SHARED((_N,), jnp.float32),
            pltpu.VMEM_SHARED((_N,), jnp.float32),
        ],
        compiler_params=pltpu.CompilerParams(needs_layout_passes=False),
    )
    def attn(edge_h, q0_h, k0_h, v0_h, q1_h, k1_h, v1_h, out0_h, out1_h,
             sa0, da0, qra0, kra0, vra0, gsa0, ssa0, isa0,
             sa1, da1, qra1, kra1, vra1, gsa1, ssa1, isa1,
             accum, dacc0, dacc1):
        cid = lax.axis_index("c")
        sid = lax.axis_index("s")
        lane = lax.iota(jnp.int32, _L)
        zeros = jnp.zeros((_L,), jnp.float32)
        sets = (
            (sa0, da0, qra0, kra0, vra0, gsa0, ssa0, isa0),
            (sa1, da1, qra1, kra1, vra1, gsa1, ssa1, isa1),
        )

        for r in range(_RB):
            for j in range(8):
                qra0[r, j * _L:(j + 1) * _L] = zeros

        def _zero_blk(t, _):
            b = sid + _NS * t

            @pl.when(b < _NB)
            def _():
                pltpu.sync_copy(qra0.at[pl.ds(0, _RB)], accum.at[pl.ds(b * _RB, _RB)])
                pltpu.sync_copy(qra0.at[0, pl.ds(0, _RB)], dacc0.at[pl.ds(b * _RB, _RB)])
                pltpu.sync_copy(qra0.at[0, pl.ds(0, _RB)], dacc1.at[pl.ds(b * _RB, _RB)])

            return _

        lax.fori_loop(0, _BPS, _zero_blk, None)
        plsc.subcore_barrier()

        def _run(qt, kt, vt):
            def fire_idx(S, i):
                src_v, dst_v, isem = S[0], S[1], S[7]
                base = (sid + _NS * i) * _CH
                pltpu.async_copy(edge_h.at[0, pl.ds(base, _CH)], src_v, isem)
                pltpu.async_copy(edge_h.at[1, pl.ds(base, _CH)], dst_v, isem)

            def wait_idx(S):
                src_v, dst_v, isem = S[0], S[1], S[7]
                pltpu.make_async_copy(
                    edge_h.at[0, pl.ds(0, _CH)], src_v, isem).wait()
                pltpu.make_async_copy(
                    edge_h.at[0, pl.ds(0, _CH)], dst_v, isem).wait()

            def fire_gather(S):
                src_v, dst_v, qr, kr, vr, gsem = S[:6]
                pltpu.async_copy(qt.at[src_v], qr, gsem)
                pltpu.async_copy(kt.at[dst_v], kr, gsem)
                pltpu.async_copy(vt.at[dst_v], vr, gsem)

            def drain_gather(S):
                src_v, dst_v, qr, kr, vr, gsem = S[:6]
                pltpu.make_async_copy(qt.at[src_v], qr, gsem).wait()
                pltpu.make_async_copy(kt.at[dst_v], kr, gsem).wait()
                pltpu.make_async_copy(vt.at[dst_v], vr, gsem).wait()

            def fire_scatter(S):
                src_v, _, qr, kr, vr, _, ssem = S[:7]
                pltpu.async_copy(vr, accum.at[src_v], ssem, add=True)
                pltpu.async_copy(
                    qr.at[0, pl.ds(0, _CH)], dacc0.at[src_v], ssem, add=True)
                pltpu.async_copy(
                    kr.at[0, pl.ds(0, _CH)], dacc1.at[src_v], ssem, add=True)

            def drain_scatter(S):
                src_v, _, qr, kr, vr, _, ssem = S[:7]
                pltpu.make_async_copy(vr, accum.at[src_v], ssem).wait()
                pltpu.make_async_copy(
                    qr.at[0, pl.ds(0, _CH)], dacc0.at[src_v], ssem).wait()
                pltpu.make_async_copy(
                    kr.at[0, pl.ds(0, _CH)], dacc1.at[src_v], ssem).wait()

            def compute(S):
                _, _, qr, kr, vr = S[:5]

                def _edge(e, _):
                    acc0 = qr[e, 0:_L] * kr[e, 0:_L]
                    acc1 = qr[e, 4 * _L:5 * _L] * kr[e, 4 * _L:5 * _L]
                    for j in range(1, 4):
                        acc0 = acc0 + qr[e, j * _L:(j + 1) * _L] * kr[e, j * _L:(j + 1) * _L]
                        jj = j + 4
                        acc1 = acc1 + qr[e, jj * _L:(jj + 1) * _L] * kr[e, jj * _L:(jj + 1) * _L]
                    w0 = jnp.exp(jnp.full((_L,), jnp.sum(acc0), jnp.float32))
                    w1 = jnp.exp(jnp.full((_L,), jnp.sum(acc1), jnp.float32))
                    for j in range(4):
                        vr[e, j * _L:(j + 1) * _L] = w0 * vr[e, j * _L:(j + 1) * _L]
                    for j in range(4, 8):
                        vr[e, j * _L:(j + 1) * _L] = w1 * vr[e, j * _L:(j + 1) * _L]
                    plsc.store_compressed(qr.at[0, pl.ds(e, _L)], w0, mask=lane == 0)
                    plsc.store_compressed(kr.at[0, pl.ds(e, _L)], w1, mask=lane == 0)
                    return _

                lax.fori_loop(0, _CH, _edge, None, unroll=4)

            fire_idx(sets[0], 0)
            wait_idx(sets[0])
            fire_gather(sets[0])

            def _pair(t, _):
                for b in (0, 1):
                    S = sets[b]
                    T = sets[1 - b]
                    i = 2 * t + b
                    g = sid + _NS * i

                    @pl.when(g < _G)
                    def _():
                        drain_gather(S)

                    @pl.when(sid + _NS * (i + 1) < _G)
                    def _():
                        @pl.when(i >= 1)
                        def _():
                            drain_scatter(T)

                        fire_idx(T, i + 1)
                        wait_idx(T)
                        fire_gather(T)

                    @pl.when(g < _G)
                    def _():
                        compute(S)
                        fire_scatter(S)

                return _

            lax.fori_loop(0, _PAIRS, _pair, None)
            drain_scatter(sets[0])
            drain_scatter(sets[1])

        @pl.when(cid == 0)
        def _():
            _run(q0_h, k0_h, v0_h)

        @pl.when(cid == 1)
        def _():
            _run(q1_h, k1_h, v1_h)

        plsc.subcore_barrier()

        def _fin(t, _):
            b = sid + _NS * t

            @pl.when(b < _NB)
            def _():
                pltpu.sync_copy(accum.at[pl.ds(b * _RB, _RB)], qra0.at[pl.ds(0, _RB)])
                pltpu.sync_copy(dacc0.at[pl.ds(b * _RB, _RB)], qra0.at[16, pl.ds(0, _RB)])
                pltpu.sync_copy(dacc1.at[pl.ds(b * _RB, _RB)], qra0.at[16, pl.ds(_L, _RB)])
                dv0 = 1.0 / (qra0[16, 0:_L] + 1e-16)
                dv1 = 1.0 / (qra0[16, _L:2 * _L] + 1e-16)
                for r in range(_RB):
                    i0 = jnp.full((_L,), dv0[r], jnp.float32)
                    i1 = jnp.full((_L,), dv1[r], jnp.float32)
                    for j in range(4):
                        kra0[r, j * _L:(j + 1) * _L] = qra0[r, j * _L:(j + 1) * _L] * i0
                    for j in range(4, 8):
                        kra0[r, j * _L:(j + 1) * _L] = qra0[r, j * _L:(j + 1) * _L] * i1

                @pl.when(cid == 0)
                def _():
                    pltpu.sync_copy(kra0.at[pl.ds(0, _RB)],
                                    out0_h.at[pl.ds(b * _RB, _RB)])

                @pl.when(cid == 1)
                def _():
                    pltpu.sync_copy(kra0.at[pl.ds(0, _RB)],
                                    out1_h.at[pl.ds(b * _RB, _RB)])

            return _

        lax.fori_loop(0, _BPS, _fin, None)

    return attn(edge, q0, k0, v0, q1, k1, v1)


def kernel(x, edge, Qw, Qb, Kw, Kb, Vw, Vb):
    scale = 1.0 / (_DK ** 0.5)
    wcat = jnp.concatenate([Qw, Kw * scale, Vw], axis=1)
    bcat = jnp.concatenate([Qb, Kb * scale, Vb]).reshape(1, 3 * _D)
    qkv = _project(x, wcat, bcat)
    q0 = qkv[:, 0:128]
    q1 = qkv[:, 128:256]
    k0 = qkv[:, 256:384]
    k1 = qkv[:, 384:512]
    v0 = qkv[:, 512:640]
    v1 = qkv[:, 640:768]
    o0, o1 = _sc_attention(edge, q0, k0, v0, q1, k1, v1)
    return jnp.concatenate([o0, o1], axis=1)
